# Initial kernel scaffold; baseline (speedup 1.0000x reference)
import jax
import jax.numpy as jnp
from jax.experimental import pallas as pl


def kernel(edge_index, u_table, i_table, W_gc, b_gc, W_bi, b_bi):
    raise NotImplementedError("write your pallas kernel here")


# speedup vs baseline: 2.5140x; 2.5140x over previous
import functools

import jax
import jax.numpy as jnp
from jax import lax
from jax.experimental import pallas as pl
from jax.experimental.pallas import tpu as pltpu
from jax.experimental.pallas import tpu_sc as plsc

NV = 50000
NP = 50176
D = 64
DH = 16
NQ = 4
NL = 3
E = 800000
EP = 802816
ER = EP // 128
NSUB = 16
VPT = NP // NSUB
EPT = ER // NSUB
GRP = EPT // 8
BLK = 512
NB = NP // BLK

_mesh = plsc.VectorSubcoreMesh(
    core_axis_name="c", subcore_axis_name="s", num_cores=2, num_subcores=16
)
_sc_params = pltpu.CompilerParams(use_tc_tiling_on_sc=False)

_f32 = jnp.float32
_i32 = jnp.int32


@functools.partial(
    pl.kernel,
    out_type=jax.ShapeDtypeStruct((2 * NP, 16), _f32),
    mesh=_mesh,
    compiler_params=_sc_params,
    scratch_types=[
        pltpu.VMEM((8, 128), _i32),
        pltpu.VMEM((128, 16), _f32),
        pltpu.VMEM((1568, 16), _f32),
        pltpu.VMEM_SHARED((NP, 16), _f32),
    ],
)
def _deg_kernel(eraw, deg_out, idxv, onesv, zbuf, acc):
    c = lax.axis_index("c")
    s = lax.axis_index("s")
    one = jnp.ones((16,), _f32)
    z = jnp.zeros((16,), _f32)

    def fill_ones(i, _):
        onesv[i, 0:16] = one
        return 0

    lax.fori_loop(0, 128, fill_ones, 0)

    def fill_zero(i, _):
        zbuf[i, 0:16] = z
        return 0

    lax.fori_loop(0, 1568, fill_zero, 0)

    vbase = s * VPT
    pltpu.sync_copy(zbuf, acc.at[pl.ds(vbase, 1568)])
    pltpu.sync_copy(zbuf, acc.at[pl.ds(vbase + 1568, 1568)])
    plsc.subcore_barrier()

    row0 = c * ER + s * EPT

    def group(g, _):
        pltpu.sync_copy(eraw.at[pl.ds(row0 + g * 8, 8)], idxv)
        for j in range(8):
            pltpu.sync_copy(onesv, acc.at[idxv.at[j]], add=True)
        return 0

    lax.fori_loop(0, GRP, group, 0)
    plsc.subcore_barrier()
    pltpu.sync_copy(acc.at[pl.ds(vbase, VPT)], deg_out.at[pl.ds(c * NP + vbase, VPT)])


def _make_sc_pass(swap: bool):
    @functools.partial(
        pl.kernel,
        out_type=tuple(
            jax.ShapeDtypeStruct((2 * NP, DH), _f32) for _ in range(NQ)
        ),
        mesh=_mesh,
        compiler_params=_sc_params,
        scratch_types=[
            pltpu.VMEM((8, 128), _i32),
            pltpu.VMEM((8, 128), _i32),
            pltpu.VMEM((128, DH), _f32),
            pltpu.VMEM((1568, DH), _f32),
            pltpu.VMEM_SHARED((NP, DH), _f32),
        ],
    )
    def _sc_pass(eoff, eraw, t0, t1, t2, t3, o0, o1, o2, o3,
                 gidxv, sidxv, rowsv, zbuf, acc):
        c = lax.axis_index("c")
        s = lax.axis_index("s")
        gside = (1 - c) if swap else c
        sside = 1 - gside
        z = jnp.zeros((16,), _f32)

        def fill_zero(i, _):
            zbuf[i, 0:16] = z
            return 0

        lax.fori_loop(0, 1568, fill_zero, 0)
        vbase = s * VPT
        r0 = s * EPT
        for tbl, out in ((t0, o0), (t1, o1), (t2, o2), (t3, o3)):
            pltpu.sync_copy(zbuf, acc.at[pl.ds(vbase, 1568)])
            pltpu.sync_copy(zbuf, acc.at[pl.ds(vbase + 1568, 1568)])
            plsc.subcore_barrier()

            def group(g, _):
                pltpu.sync_copy(eoff.at[pl.ds(gside * ER + r0 + g * 8, 8)], gidxv)
                pltpu.sync_copy(eraw.at[pl.ds(sside * ER + r0 + g * 8, 8)], sidxv)
                for j in range(8):
                    pltpu.sync_copy(tbl.at[gidxv.at[j]], rowsv)
                    pltpu.sync_copy(rowsv, acc.at[sidxv.at[j]], add=True)
                return 0

            lax.fori_loop(0, GRP, group, 0)
            plsc.subcore_barrier()
            pltpu.sync_copy(
                acc.at[pl.ds(vbase, VPT)], out.at[pl.ds(sside * NP + vbase, VPT)]
            )

    return _sc_pass


_sc_pass_a = _make_sc_pass(swap=False)
_sc_pass_b = _make_sc_pass(swap=True)


def _leaky(x):
    return jnp.where(x >= 0, x, 0.2 * x)


def _q_specs():
    return [pl.BlockSpec((1, BLK, DH), lambda t, b: (t, b, 0)) for _ in range(NQ)]


def _q_shapes():
    return [jax.ShapeDtypeStruct((2, NP, DH), _f32) for _ in range(NQ)]


def _prep_body(deg_ref, all0_ref, s_ref, d_ref, *xq_refs):
    dg = deg_ref[0]
    pos = dg > 0
    dsafe = jnp.maximum(dg, 1e-12)
    sv = jnp.where(pos, lax.rsqrt(dsafe), 0.0)
    dv = jnp.where(pos, 1.0 / dsafe, 0.0)
    s_ref[0] = sv
    d_ref[0] = dv
    xs = all0_ref[0] * sv
    for q in range(NQ):
        xq_refs[q][0] = xs[:, q * DH:(q + 1) * DH]


_tc_prep = pl.pallas_call(
    _prep_body,
    grid=(2, NB),
    in_specs=[
        pl.BlockSpec((1, BLK, 1), lambda t, b: (t, b, 0)),
        pl.BlockSpec((1, BLK, D), lambda t, b: (t, b, 0)),
    ],
    out_specs=[
        pl.BlockSpec((1, BLK, 1), lambda t, b: (t, b, 0)),
        pl.BlockSpec((1, BLK, 1), lambda t, b: (t, b, 0)),
    ] + _q_specs(),
    out_shape=[
        jax.ShapeDtypeStruct((2, NP, 1), _f32),
        jax.ShapeDtypeStruct((2, NP, 1), _f32),
    ] + _q_shapes(),
)


def _scale_y_body(y0, y1, y2, y3, d_ref, z0, z1, z2, z3):
    dv = d_ref[0]
    z0[0] = y0[0] * dv
    z1[0] = y1[0] * dv
    z2[0] = y2[0] * dv
    z3[0] = y3[0] * dv


_tc_scale_y = pl.pallas_call(
    _scale_y_body,
    grid=(2, NB),
    in_specs=_q_specs() + [pl.BlockSpec((1, BLK, 1), lambda t, b: (t, b, 0))],
    out_specs=_q_specs(),
    out_shape=_q_shapes(),
)


def _make_dense(last: bool):
    def body(all_ref, acc_ref, g0, g1, g2, g3, s_ref, wgc_ref, bgc_ref,
             wbi_ref, bbi_ref, allo_ref, acco_ref, *xq_refs):
        al = all_ref[0]
        sv = s_ref[0]
        g = jnp.concatenate([g0[0], g1[0], g2[0], g3[0]], axis=1) * sv
        sum_e = _leaky(
            jnp.dot(g, wgc_ref[...], preferred_element_type=_f32)
            + bgc_ref[...] + al
        )
        bi = _leaky(
            jnp.dot(al * g, wbi_ref[...], preferred_element_type=_f32)
            + bbi_ref[...]
        )
        new = sum_e + bi
        norm = jnp.sqrt(jnp.sum(new * new, axis=1, keepdims=True))
        new = new / jnp.maximum(norm, 1e-12)
        allo_ref[0] = new
        acc_o = acc_ref[0] + new
        if last:
            acc_o = acc_o * 0.25
        acco_ref[0] = acc_o
        xs = new * sv
        for q in range(NQ):
            xq_refs[q][0] = xs[:, q * DH:(q + 1) * DH]

    return pl.pallas_call(
        body,
        grid=(2, NB),
        in_specs=[
            pl.BlockSpec((1, BLK, D), lambda t, b: (t, b, 0)),
            pl.BlockSpec((1, BLK, D), lambda t, b: (t, b, 0)),
        ] + _q_specs() + [
            pl.BlockSpec((1, BLK, 1), lambda t, b: (t, b, 0)),
            pl.BlockSpec((D, D), lambda t, b: (0, 0)),
            pl.BlockSpec((1, D), lambda t, b: (0, 0)),
            pl.BlockSpec((D, D), lambda t, b: (0, 0)),
            pl.BlockSpec((1, D), lambda t, b: (0, 0)),
        ],
        out_specs=[
            pl.BlockSpec((1, BLK, D), lambda t, b: (t, b, 0)),
            pl.BlockSpec((1, BLK, D), lambda t, b: (t, b, 0)),
        ] + _q_specs(),
        out_shape=[
            jax.ShapeDtypeStruct((2, NP, D), _f32),
            jax.ShapeDtypeStruct((2, NP, D), _f32),
        ] + _q_shapes(),
    )


_tc_dense = _make_dense(last=False)
_tc_dense_last = _make_dense(last=True)


def kernel(edge_index, u_table, i_table, W_gc, b_gc, W_bi, b_bi):
    users = edge_index[0]
    items = edge_index[1]
    pad_e = EP - E
    users_p = jnp.concatenate([users, jnp.full((pad_e,), NV, _i32)]).reshape(ER, 128)
    items_p = jnp.concatenate([items, jnp.full((pad_e,), NV, _i32)]).reshape(ER, 128)
    eraw = jnp.concatenate([users_p, items_p], axis=0)
    eoff = jnp.concatenate([users_p, items_p + NP], axis=0)

    pad_v = NP - NV
    up = jnp.concatenate([u_table, jnp.zeros((pad_v, D), _f32)], axis=0)
    ip = jnp.concatenate([i_table, jnp.zeros((pad_v, D), _f32)], axis=0)
    all0 = jnp.stack([up, ip], axis=0)

    wgcT = jnp.swapaxes(W_gc, 1, 2)
    wbiT = jnp.swapaxes(W_bi, 1, 2)

    deg16 = _deg_kernel(eraw)
    deg3 = deg16[:, :1].reshape(2, NP, 1)
    s3, d3, *xq = _tc_prep(deg3, all0)

    flat = lambda a: a.reshape(2 * NP, DH)
    unflat = lambda a: a.reshape(2, NP, DH)

    all_e = all0
    acc = all0
    for l in range(NL):
        yq = _sc_pass_a(eoff, eraw, *[flat(x) for x in xq])
        zq = _tc_scale_y(*[unflat(y) for y in yq], d3)
        gq = _sc_pass_b(eoff, eraw, *[flat(z) for z in zq])
        dense = _tc_dense_last if l == NL - 1 else _tc_dense
        all_e, acc, *xq = dense(
            all_e, acc, *[unflat(g) for g in gq], s3,
            wgcT[l], b_gc[l:l + 1], wbiT[l], b_bi[l:l + 1],
        )
    return acc[0, :NV], acc[1, :NV]


# speedup vs baseline: 5.3062x; 2.1106x over previous
import functools

import jax
import jax.numpy as jnp
from jax import lax
from jax.experimental import pallas as pl
from jax.experimental.pallas import tpu as pltpu
from jax.experimental.pallas import tpu_sc as plsc

NV = 50000
NP = 50176
D = 64
DH = 16
NQ = 4
NL = 3
E = 800000
EP = 802816
ER = EP // 128
NSUB = 16
VPT = NP // NSUB
EPT = ER // NSUB
GRP = EPT // 8
BLK = 512
NB = NP // BLK

_mesh = plsc.VectorSubcoreMesh(
    core_axis_name="c", subcore_axis_name="s", num_cores=2, num_subcores=16
)
_sc_params = pltpu.CompilerParams(use_tc_tiling_on_sc=False)

_f32 = jnp.float32
_i32 = jnp.int32


@functools.partial(
    pl.kernel,
    out_type=jax.ShapeDtypeStruct((2 * NP, 16), _f32),
    mesh=_mesh,
    compiler_params=_sc_params,
    scratch_types=[
        pltpu.VMEM((8, 128), _i32),
        pltpu.VMEM((128, 16), _f32),
        pltpu.VMEM((1568, 16), _f32),
        pltpu.VMEM_SHARED((NP, 16), _f32),
    ],
)
def _deg_kernel(eraw, deg_out, idxv, onesv, zbuf, acc):
    c = lax.axis_index("c")
    s = lax.axis_index("s")
    one = jnp.ones((16,), _f32)
    z = jnp.zeros((16,), _f32)

    def fill_ones(i, _):
        onesv[i, 0:16] = one
        return 0

    lax.fori_loop(0, 128, fill_ones, 0)

    def fill_zero(i, _):
        zbuf[i, 0:16] = z
        return 0

    lax.fori_loop(0, 1568, fill_zero, 0)

    vbase = s * VPT
    pltpu.sync_copy(zbuf, acc.at[pl.ds(vbase, 1568)])
    pltpu.sync_copy(zbuf, acc.at[pl.ds(vbase + 1568, 1568)])
    plsc.subcore_barrier()

    row0 = c * ER + s * EPT

    def group(g, _):
        pltpu.sync_copy(eraw.at[pl.ds(row0 + g * 8, 8)], idxv)
        for j in range(8):
            pltpu.sync_copy(onesv, acc.at[idxv.at[j]], add=True)
        return 0

    lax.fori_loop(0, GRP, group, 0)
    plsc.subcore_barrier()
    pltpu.sync_copy(acc.at[pl.ds(vbase, VPT)], deg_out.at[pl.ds(c * NP + vbase, VPT)])


def _make_sc_pass(swap: bool):
    @functools.partial(
        pl.kernel,
        out_type=tuple(
            jax.ShapeDtypeStruct((2 * NP, DH), _f32) for _ in range(NQ)
        ),
        mesh=_mesh,
        compiler_params=_sc_params,
        scratch_types=[
            pltpu.VMEM((3, 8, 128), _i32),
            pltpu.VMEM((3, 8, 128), _i32),
            pltpu.VMEM((16, 128, DH), _f32),
            pltpu.VMEM((1568, DH), _f32),
            pltpu.VMEM_SHARED((NP, DH), _f32),
            pltpu.SemaphoreType.DMA,
            pltpu.SemaphoreType.DMA,
            pltpu.SemaphoreType.DMA,
            pltpu.SemaphoreType.DMA,
        ],
    )
    def _sc_pass(eoff, eraw, t0, t1, t2, t3, o0, o1, o2, o3,
                 gidxv, sidxv, rbufs, zbuf, acc, isem, gsem, ssem0, ssem1):
        c = lax.axis_index("c")
        s = lax.axis_index("s")
        gside = (1 - c) if swap else c
        sside = 1 - gside
        z = jnp.zeros((16,), _f32)

        def fill_zero(i, _):
            zbuf[i, 0:16] = z
            return 0

        lax.fori_loop(0, 1568, fill_zero, 0)
        vbase = s * VPT
        r0 = s * EPT

        def stage_idx(g, slot):
            pltpu.async_copy(
                eoff.at[pl.ds(gside * ER + r0 + g * 8, 8)], gidxv.at[slot], isem
            )
            pltpu.async_copy(
                eraw.at[pl.ds(sside * ER + r0 + g * 8, 8)], sidxv.at[slot], isem
            )

        def wait_idx(slot):
            pltpu.make_async_copy(
                eoff.at[pl.ds(r0, 8)], gidxv.at[slot], isem
            ).wait()
            pltpu.make_async_copy(
                eraw.at[pl.ds(r0, 8)], sidxv.at[slot], isem
            ).wait()

        for tbl, out in ((t0, o0), (t1, o1), (t2, o2), (t3, o3)):
            pltpu.sync_copy(zbuf, acc.at[pl.ds(vbase, 1568)])
            pltpu.sync_copy(zbuf, acc.at[pl.ds(vbase + 1568, 1568)])
            plsc.subcore_barrier()
            stage_idx(0, 0)

            def group(g, _):
                slot = lax.rem(g, 3)
                p = lax.rem(g, 2)
                wait_idx(slot)

                @pl.when(g >= 2)
                def _():
                    for sem_i, ssem in ((0, ssem0), (1, ssem1)):
                        @pl.when(p == sem_i)
                        def _():
                            for j in range(8):
                                pltpu.make_async_copy(
                                    rbufs.at[j], acc.at[sidxv.at[slot, 0]], ssem
                                ).wait()

                @pl.when(g < GRP - 1)
                def _():
                    stage_idx(g + 1, lax.rem(g + 1, 3))

                for j in range(8):
                    pltpu.async_copy(
                        tbl.at[gidxv.at[slot, j]], rbufs.at[p * 8 + j], gsem
                    )
                for j in range(8):
                    pltpu.make_async_copy(
                        tbl.at[gidxv.at[slot, 0]], rbufs.at[j], gsem
                    ).wait()
                for sem_i, ssem in ((0, ssem0), (1, ssem1)):
                    @pl.when(p == sem_i)
                    def _():
                        for j in range(8):
                            pltpu.async_copy(
                                rbufs.at[p * 8 + j], acc.at[sidxv.at[slot, j]],
                                ssem, add=True,
                            )
                return 0

            lax.fori_loop(0, GRP, group, 0)
            for ssem in (ssem0, ssem1):
                for j in range(8):
                    pltpu.make_async_copy(
                        rbufs.at[j], acc.at[sidxv.at[0, 0]], ssem
                    ).wait()
            plsc.subcore_barrier()
            pltpu.sync_copy(
                acc.at[pl.ds(vbase, VPT)], out.at[pl.ds(sside * NP + vbase, VPT)]
            )

    return _sc_pass


_sc_pass_a = _make_sc_pass(swap=False)
_sc_pass_b = _make_sc_pass(swap=True)


def _leaky(x):
    return jnp.where(x >= 0, x, 0.2 * x)


def _q_specs():
    return [pl.BlockSpec((1, BLK, DH), lambda t, b: (t, b, 0)) for _ in range(NQ)]


def _q_shapes():
    return [jax.ShapeDtypeStruct((2, NP, DH), _f32) for _ in range(NQ)]


def _prep_body(deg_ref, all0_ref, s_ref, d_ref, *xq_refs):
    dg = deg_ref[0]
    pos = dg > 0
    dsafe = jnp.maximum(dg, 1e-12)
    sv = jnp.where(pos, lax.rsqrt(dsafe), 0.0)
    dv = jnp.where(pos, 1.0 / dsafe, 0.0)
    s_ref[0] = sv
    d_ref[0] = dv
    xs = all0_ref[0] * sv
    for q in range(NQ):
        xq_refs[q][0] = xs[:, q * DH:(q + 1) * DH]


_tc_prep = pl.pallas_call(
    _prep_body,
    grid=(2, NB),
    in_specs=[
        pl.BlockSpec((1, BLK, 1), lambda t, b: (t, b, 0)),
        pl.BlockSpec((1, BLK, D), lambda t, b: (t, b, 0)),
    ],
    out_specs=[
        pl.BlockSpec((1, BLK, 1), lambda t, b: (t, b, 0)),
        pl.BlockSpec((1, BLK, 1), lambda t, b: (t, b, 0)),
    ] + _q_specs(),
    out_shape=[
        jax.ShapeDtypeStruct((2, NP, 1), _f32),
        jax.ShapeDtypeStruct((2, NP, 1), _f32),
    ] + _q_shapes(),
)


def _scale_y_body(y0, y1, y2, y3, d_ref, z0, z1, z2, z3):
    dv = d_ref[0]
    z0[0] = y0[0] * dv
    z1[0] = y1[0] * dv
    z2[0] = y2[0] * dv
    z3[0] = y3[0] * dv


_tc_scale_y = pl.pallas_call(
    _scale_y_body,
    grid=(2, NB),
    in_specs=_q_specs() + [pl.BlockSpec((1, BLK, 1), lambda t, b: (t, b, 0))],
    out_specs=_q_specs(),
    out_shape=_q_shapes(),
)


def _make_dense(last: bool):
    def body(all_ref, acc_ref, g0, g1, g2, g3, s_ref, wgc_ref, bgc_ref,
             wbi_ref, bbi_ref, allo_ref, acco_ref, *xq_refs):
        al = all_ref[0]
        sv = s_ref[0]
        g = jnp.concatenate([g0[0], g1[0], g2[0], g3[0]], axis=1) * sv
        sum_e = _leaky(
            jnp.dot(g, wgc_ref[...], preferred_element_type=_f32)
            + bgc_ref[...] + al
        )
        bi = _leaky(
            jnp.dot(al * g, wbi_ref[...], preferred_element_type=_f32)
            + bbi_ref[...]
        )
        new = sum_e + bi
        norm = jnp.sqrt(jnp.sum(new * new, axis=1, keepdims=True))
        new = new / jnp.maximum(norm, 1e-12)
        allo_ref[0] = new
        acc_o = acc_ref[0] + new
        if last:
            acc_o = acc_o * 0.25
        acco_ref[0] = acc_o
        xs = new * sv
        for q in range(NQ):
            xq_refs[q][0] = xs[:, q * DH:(q + 1) * DH]

    return pl.pallas_call(
        body,
        grid=(2, NB),
        in_specs=[
            pl.BlockSpec((1, BLK, D), lambda t, b: (t, b, 0)),
            pl.BlockSpec((1, BLK, D), lambda t, b: (t, b, 0)),
        ] + _q_specs() + [
            pl.BlockSpec((1, BLK, 1), lambda t, b: (t, b, 0)),
            pl.BlockSpec((D, D), lambda t, b: (0, 0)),
            pl.BlockSpec((1, D), lambda t, b: (0, 0)),
            pl.BlockSpec((D, D), lambda t, b: (0, 0)),
            pl.BlockSpec((1, D), lambda t, b: (0, 0)),
        ],
        out_specs=[
            pl.BlockSpec((1, BLK, D), lambda t, b: (t, b, 0)),
            pl.BlockSpec((1, BLK, D), lambda t, b: (t, b, 0)),
        ] + _q_specs(),
        out_shape=[
            jax.ShapeDtypeStruct((2, NP, D), _f32),
            jax.ShapeDtypeStruct((2, NP, D), _f32),
        ] + _q_shapes(),
    )


_tc_dense = _make_dense(last=False)
_tc_dense_last = _make_dense(last=True)


def kernel(edge_index, u_table, i_table, W_gc, b_gc, W_bi, b_bi):
    users = edge_index[0]
    items = edge_index[1]
    pad_e = EP - E
    users_p = jnp.concatenate([users, jnp.full((pad_e,), NV, _i32)]).reshape(ER, 128)
    items_p = jnp.concatenate([items, jnp.full((pad_e,), NV, _i32)]).reshape(ER, 128)
    eraw = jnp.concatenate([users_p, items_p], axis=0)
    eoff = jnp.concatenate([users_p, items_p + NP], axis=0)

    pad_v = NP - NV
    up = jnp.concatenate([u_table, jnp.zeros((pad_v, D), _f32)], axis=0)
    ip = jnp.concatenate([i_table, jnp.zeros((pad_v, D), _f32)], axis=0)
    all0 = jnp.stack([up, ip], axis=0)

    wgcT = jnp.swapaxes(W_gc, 1, 2)
    wbiT = jnp.swapaxes(W_bi, 1, 2)

    deg16 = _deg_kernel(eraw)
    deg3 = deg16[:, :1].reshape(2, NP, 1)
    s3, d3, *xq = _tc_prep(deg3, all0)

    flat = lambda a: a.reshape(2 * NP, DH)
    unflat = lambda a: a.reshape(2, NP, DH)

    all_e = all0
    acc = all0
    for l in range(NL):
        yq = _sc_pass_a(eoff, eraw, *[flat(x) for x in xq])
        zq = _tc_scale_y(*[unflat(y) for y in yq], d3)
        gq = _sc_pass_b(eoff, eraw, *[flat(z) for z in zq])
        dense = _tc_dense_last if l == NL - 1 else _tc_dense
        all_e, acc, *xq = dense(
            all_e, acc, *[unflat(g) for g in gq], s3,
            wgcT[l], b_gc[l:l + 1], wbiT[l], b_bi[l:l + 1],
        )
    return acc[0, :NV], acc[1, :NV]


# speedup vs baseline: 7.0322x; 1.3253x over previous
import functools

import jax
import jax.numpy as jnp
from jax import lax
from jax.experimental import pallas as pl
from jax.experimental.pallas import tpu as pltpu
from jax.experimental.pallas import tpu_sc as plsc

NV = 50000
NP = 50176
D = 64
DH = 16
NQ = 4
NL = 3
E = 800000
EP = 802816
ER = EP // 128
NSUB = 16
VPT = NP // NSUB
EPT = ER // NSUB
GRP = EPT // 8
WBC = 784
BLK = 512
NB = NP // BLK

_mesh = plsc.VectorSubcoreMesh(
    core_axis_name="c", subcore_axis_name="s", num_cores=2, num_subcores=16
)
_sc_params = pltpu.CompilerParams(use_tc_tiling_on_sc=False)

_f32 = jnp.float32
_i32 = jnp.int32


@functools.partial(
    pl.kernel,
    out_type=(
        jax.ShapeDtypeStruct((2 * NP, 16), _f32),
        jax.ShapeDtypeStruct((2 * NP, 16), _f32),
    ),
    mesh=_mesh,
    compiler_params=_sc_params,
    scratch_types=[
        pltpu.VMEM((8, 128), _i32),
        pltpu.VMEM((128, 16), _f32),
        pltpu.VMEM((1568, 16), _f32),
        pltpu.VMEM((WBC, 16), _f32),
        pltpu.VMEM_SHARED((NP, 16), _f32),
    ],
)
def _deg_kernel(eraw, deg_out, dinv_out, idxv, onesv, zbuf, dbuf, acc):
    c = lax.axis_index("c")
    s = lax.axis_index("s")
    one = jnp.ones((16,), _f32)
    z = jnp.zeros((16,), _f32)

    def fill_ones(i, _):
        onesv[i, 0:16] = one
        return 0

    lax.fori_loop(0, 128, fill_ones, 0)

    def fill_zero(i, _):
        zbuf[i, 0:16] = z
        return 0

    lax.fori_loop(0, 1568, fill_zero, 0)

    vbase = s * VPT
    pltpu.sync_copy(zbuf, acc.at[pl.ds(vbase, 1568)])
    pltpu.sync_copy(zbuf, acc.at[pl.ds(vbase + 1568, 1568)])
    plsc.subcore_barrier()

    row0 = c * ER + s * EPT

    def group(g, _):
        pltpu.sync_copy(eraw.at[pl.ds(row0 + g * 8, 8)], idxv)
        for j in range(8):
            pltpu.sync_copy(onesv, acc.at[idxv.at[j]], add=True)
        return 0

    lax.fori_loop(0, GRP, group, 0)
    plsc.subcore_barrier()
    pltpu.sync_copy(acc.at[pl.ds(vbase, VPT)], deg_out.at[pl.ds(c * NP + vbase, VPT)])
    for ch in range(VPT // WBC):
        base = vbase + ch * WBC
        pltpu.sync_copy(acc.at[pl.ds(base, WBC)], dbuf)

        def recip(i, _):
            dg = dbuf[i, 0:16]
            dv = jnp.where(dg > 0, 1.0 / jnp.maximum(dg, 1e-12), 0.0)
            dbuf[i, 0:16] = dv
            return 0

        lax.fori_loop(0, WBC, recip, 0)
        pltpu.sync_copy(dbuf, dinv_out.at[pl.ds(c * NP + base, WBC)])


def _make_sc_pass(swap: bool, scale_wb: bool):
    @functools.partial(
        pl.kernel,
        out_type=jax.ShapeDtypeStruct((2 * NP, NQ, DH), _f32),
        mesh=_mesh,
        compiler_params=_sc_params,
        scratch_types=[
            pltpu.VMEM((3, 8, 128), _i32),
            pltpu.VMEM((3, 8, 128), _i32),
            pltpu.VMEM((16, 128, DH), _f32),
            pltpu.VMEM((1568, DH), _f32),
            pltpu.VMEM_SHARED((NP, DH), _f32),
            pltpu.SemaphoreType.DMA,
            pltpu.SemaphoreType.DMA,
            pltpu.SemaphoreType.DMA,
            pltpu.SemaphoreType.DMA,
        ],
    )
    def _sc_pass(eoffq, eraw, tbl, dinv, out,
                 gidxv, sidxv, rbufs, zbuf, acc,
                 isem, gsem, ssem0, ssem1):
        c = lax.axis_index("c")
        s = lax.axis_index("s")
        gside = (1 - c) if swap else c
        sside = 1 - gside
        z = jnp.zeros((16,), _f32)

        def fill_zero(i, _):
            zbuf[i, 0:16] = z
            return 0

        lax.fori_loop(0, 1568, fill_zero, 0)
        vbase = s * VPT
        r0 = s * EPT

        for q in range(NQ):
            def stage_idx(g, slot):
                pltpu.async_copy(
                    eoffq.at[q, pl.ds(gside * ER + r0 + g * 8, 8)],
                    gidxv.at[slot], isem,
                )
                pltpu.async_copy(
                    eraw.at[pl.ds(sside * ER + r0 + g * 8, 8)],
                    sidxv.at[slot], isem,
                )

            def wait_idx(slot):
                pltpu.make_async_copy(
                    eoffq.at[q, pl.ds(r0, 8)], gidxv.at[slot], isem
                ).wait()
                pltpu.make_async_copy(
                    eraw.at[pl.ds(r0, 8)], sidxv.at[slot], isem
                ).wait()

            pltpu.sync_copy(zbuf, acc.at[pl.ds(vbase, 1568)])
            pltpu.sync_copy(zbuf, acc.at[pl.ds(vbase + 1568, 1568)])
            plsc.subcore_barrier()
            stage_idx(0, 0)

            def group(g, _):
                slot = lax.rem(g, 3)
                p = lax.rem(g, 2)
                wait_idx(slot)

                @pl.when(g >= 2)
                def _():
                    for sem_i, ssem in ((0, ssem0), (1, ssem1)):
                        @pl.when(p == sem_i)
                        def _():
                            for j in range(8):
                                pltpu.make_async_copy(
                                    rbufs.at[j], acc.at[sidxv.at[slot, 0]], ssem
                                ).wait()

                @pl.when(g < GRP - 1)
                def _():
                    stage_idx(g + 1, lax.rem(g + 1, 3))

                for j in range(8):
                    pltpu.async_copy(
                        tbl.at[gidxv.at[slot, j]], rbufs.at[p * 8 + j], gsem
                    )
                for j in range(8):
                    pltpu.make_async_copy(
                        tbl.at[gidxv.at[slot, 0]], rbufs.at[j], gsem
                    ).wait()
                for sem_i, ssem in ((0, ssem0), (1, ssem1)):
                    @pl.when(p == sem_i)
                    def _():
                        for j in range(8):
                            pltpu.async_copy(
                                rbufs.at[p * 8 + j], acc.at[sidxv.at[slot, j]],
                                ssem, add=True,
                            )
                return 0

            lax.fori_loop(0, GRP, group, 0)
            for ssem in (ssem0, ssem1):
                for j in range(8):
                    pltpu.make_async_copy(
                        rbufs.at[j], acc.at[sidxv.at[0, 0]], ssem
                    ).wait()
            plsc.subcore_barrier()
            if scale_wb:
                nch, rem = divmod(VPT, 128)
                for ch in range(nch + (1 if rem else 0)):
                    n = 128 if ch < nch else rem
                    base = vbase + ch * 128
                    pltpu.sync_copy(
                        acc.at[pl.ds(base, n)], rbufs.at[0, pl.ds(0, n)]
                    )
                    pltpu.sync_copy(
                        dinv.at[pl.ds(sside * NP + base, n)],
                        rbufs.at[1, pl.ds(0, n)],
                    )

                    def scale(i, _):
                        rbufs[0, i, 0:16] = rbufs[0, i, 0:16] * rbufs[1, i, 0:16]
                        return 0

                    lax.fori_loop(0, n, scale, 0)
                    pltpu.sync_copy(
                        rbufs.at[0, pl.ds(0, n)],
                        out.at[pl.ds(sside * NP + base, n), q],
                    )
            else:
                pltpu.sync_copy(
                    acc.at[pl.ds(vbase, VPT)],
                    out.at[pl.ds(sside * NP + vbase, VPT), q],
                )
            if q < NQ - 1:
                plsc.subcore_barrier()

    return _sc_pass


_sc_pass_a = _make_sc_pass(swap=False, scale_wb=True)
_sc_pass_b = _make_sc_pass(swap=True, scale_wb=False)


def _leaky(x):
    return jnp.where(x >= 0, x, 0.2 * x)


def _prep_body(deg_ref, all0_ref, s_ref, xs_ref):
    dg = deg_ref[0]
    sv = jnp.where(dg > 0, lax.rsqrt(jnp.maximum(dg, 1e-12)), 0.0)
    s_ref[0] = sv
    xs_ref[0] = all0_ref[0] * sv


_tc_prep = pl.pallas_call(
    _prep_body,
    grid=(2, NB),
    in_specs=[
        pl.BlockSpec((1, BLK, 1), lambda t, b: (t, b, 0)),
        pl.BlockSpec((1, BLK, D), lambda t, b: (t, b, 0)),
    ],
    out_specs=[
        pl.BlockSpec((1, BLK, 1), lambda t, b: (t, b, 0)),
        pl.BlockSpec((1, BLK, D), lambda t, b: (t, b, 0)),
    ],
    out_shape=[
        jax.ShapeDtypeStruct((2, NP, 1), _f32),
        jax.ShapeDtypeStruct((2, NP, D), _f32),
    ],
)


def _make_dense(last: bool):
    def body(all_ref, acc_ref, g_ref, s_ref, wgc_ref, bgc_ref,
             wbi_ref, bbi_ref, allo_ref, acco_ref, xs_ref):
        al = all_ref[0]
        sv = s_ref[0]
        g = g_ref[0] * sv
        sum_e = _leaky(
            jnp.dot(g, wgc_ref[...], preferred_element_type=_f32)
            + bgc_ref[...] + al
        )
        bi = _leaky(
            jnp.dot(al * g, wbi_ref[...], preferred_element_type=_f32)
            + bbi_ref[...]
        )
        new = sum_e + bi
        norm = jnp.sqrt(jnp.sum(new * new, axis=1, keepdims=True))
        new = new / jnp.maximum(norm, 1e-12)
        allo_ref[0] = new
        acc_o = acc_ref[0] + new
        if last:
            acc_o = acc_o * 0.25
        acco_ref[0] = acc_o
        xs_ref[0] = new * sv

    return pl.pallas_call(
        body,
        grid=(2, NB),
        in_specs=[
            pl.BlockSpec((1, BLK, D), lambda t, b: (t, b, 0)),
            pl.BlockSpec((1, BLK, D), lambda t, b: (t, b, 0)),
            pl.BlockSpec((1, BLK, D), lambda t, b: (t, b, 0)),
            pl.BlockSpec((1, BLK, 1), lambda t, b: (t, b, 0)),
            pl.BlockSpec((D, D), lambda t, b: (0, 0)),
            pl.BlockSpec((1, D), lambda t, b: (0, 0)),
            pl.BlockSpec((D, D), lambda t, b: (0, 0)),
            pl.BlockSpec((1, D), lambda t, b: (0, 0)),
        ],
        out_specs=[
            pl.BlockSpec((1, BLK, D), lambda t, b: (t, b, 0)),
            pl.BlockSpec((1, BLK, D), lambda t, b: (t, b, 0)),
            pl.BlockSpec((1, BLK, D), lambda t, b: (t, b, 0)),
        ],
        out_shape=[
            jax.ShapeDtypeStruct((2, NP, D), _f32),
            jax.ShapeDtypeStruct((2, NP, D), _f32),
            jax.ShapeDtypeStruct((2, NP, D), _f32),
        ],
    )


_tc_dense = _make_dense(last=False)
_tc_dense_last = _make_dense(last=True)


def kernel(edge_index, u_table, i_table, W_gc, b_gc, W_bi, b_bi):
    users = edge_index[0]
    items = edge_index[1]
    pad_e = EP - E
    users_p = jnp.concatenate([users, jnp.full((pad_e,), NV, _i32)]).reshape(ER, 128)
    items_p = jnp.concatenate([items, jnp.full((pad_e,), NV, _i32)]).reshape(ER, 128)
    eraw = jnp.concatenate([users_p, items_p], axis=0)
    eoff = jnp.concatenate([users_p, items_p + NP], axis=0)
    eoffq = jnp.stack([eoff * NQ + q for q in range(NQ)], axis=0)

    pad_v = NP - NV
    up = jnp.concatenate([u_table, jnp.zeros((pad_v, D), _f32)], axis=0)
    ip = jnp.concatenate([i_table, jnp.zeros((pad_v, D), _f32)], axis=0)
    all0 = jnp.stack([up, ip], axis=0)

    wgcT = jnp.swapaxes(W_gc, 1, 2)
    wbiT = jnp.swapaxes(W_bi, 1, 2)

    deg16, dinv16 = _deg_kernel(eraw)
    deg3 = deg16[:, :1].reshape(2, NP, 1)
    s3, xs = _tc_prep(deg3, all0)

    all_e = all0
    acc = all0
    for l in range(NL):
        y = _sc_pass_a(eoffq, eraw, xs.reshape(2 * NP * NQ, DH), dinv16)
        g = _sc_pass_b(eoffq, eraw, y.reshape(2 * NP * NQ, DH), dinv16)
        dense = _tc_dense_last if l == NL - 1 else _tc_dense
        all_e, acc, xs = dense(
            all_e, acc, g.reshape(2, NP, D), s3,
            wgcT[l], b_gc[l:l + 1], wbiT[l], b_bi[l:l + 1],
        )
    return acc[0, :NV], acc[1, :NV]
